# Initial kernel scaffold; baseline (speedup 1.0000x reference)
import jax
import jax.numpy as jnp
from jax.experimental import pallas as pl


def kernel(x, edge_index, W1, b1, W2, b2):
    raise NotImplementedError("write your pallas kernel here")


# speedup vs baseline: 10.6481x; 10.6481x over previous
import functools

import jax
import jax.numpy as jnp
from jax import lax
from jax.experimental import pallas as pl
from jax.experimental.pallas import tpu as pltpu
from jax.experimental.pallas import tpu_sc as plsc

NC = 2
NS = 16
NW = NC * NS
CHUNK = 128

_MESH = plsc.VectorSubcoreMesh(core_axis_name="c", subcore_axis_name="s")


def _sc_degree(dst_p, zvec, ones, n_pad, e_pad):
    per_tile = e_pad // NW
    n_chunks = per_tile // CHUNK
    per_node = n_pad // NS

    @functools.partial(
        pl.kernel,
        mesh=_MESH,
        out_type=jax.ShapeDtypeStruct((NC, n_pad), jnp.float32),
        scratch_types=[
            pltpu.VMEM((CHUNK,), jnp.int32),
            pltpu.VMEM((CHUNK,), jnp.float32),
            pltpu.VMEM((per_node,), jnp.float32),
            pltpu.VMEM_SHARED((n_pad,), jnp.float32),
        ],
    )
    def k(dst_hbm, zvec_hbm, ones_hbm, out_hbm, idx_v, ones_v, z_v, acc_sh):
        cid = lax.axis_index("c")
        sid = lax.axis_index("s")
        wid = sid * NC + cid
        pltpu.sync_copy(zvec_hbm.at[pl.ds(0, per_node)], z_v)
        pltpu.sync_copy(ones_hbm, ones_v)
        pltpu.sync_copy(z_v, acc_sh.at[pl.ds(sid * per_node, per_node)])
        plsc.subcore_barrier()

        base0 = wid * per_tile

        @pl.loop(0, n_chunks)
        def _(ci):
            base = base0 + ci * CHUNK
            pltpu.sync_copy(dst_hbm.at[pl.ds(base, CHUNK)], idx_v)
            pltpu.sync_copy(ones_v, acc_sh.at[idx_v], add=True)

        plsc.subcore_barrier()
        sl = pl.ds(sid * per_node, per_node)
        pltpu.sync_copy(acc_sh.at[sl], out_hbm.at[cid, sl])

    return k(dst_p, zvec, ones)


def _sc_edge_aggregate(hs, dinv_flat, src_p, dst_p, zrow, zvec, n_pad, e_pad, hid):
    per_tile = e_pad // NW
    n_chunks = per_tile // CHUNK
    per_node = n_pad // NS
    zr = zrow.shape[0]

    @functools.partial(
        pl.kernel,
        mesh=_MESH,
        out_type=(
            jax.ShapeDtypeStruct((NC, n_pad, hid), jnp.float32),
            jax.ShapeDtypeStruct((NC, n_pad), jnp.float32),
        ),
        scratch_types=[
            pltpu.VMEM((CHUNK,), jnp.int32),
            pltpu.VMEM((CHUNK,), jnp.int32),
            pltpu.VMEM((CHUNK, hid), jnp.float32),
            pltpu.VMEM((CHUNK,), jnp.float32),
            pltpu.VMEM((zr, hid), jnp.float32),
            pltpu.VMEM((per_node,), jnp.float32),
            pltpu.VMEM_SHARED((n_pad, hid), jnp.float32),
            pltpu.VMEM_SHARED((n_pad,), jnp.float32),
            pltpu.SemaphoreType.DMA,
            pltpu.SemaphoreType.DMA,
        ],
    )
    def k(hs_hbm, dinv_hbm, src_hbm, dst_hbm, zrow_hbm, zvec_hbm,
          agg_hbm, s_hbm,
          src_v, dst_v, rows_v, dval_v, zrow_v, zvec_v, acc_agg, acc_s,
          sem_r, sem_d):
        cid = lax.axis_index("c")
        sid = lax.axis_index("s")
        wid = sid * NC + cid

        pltpu.sync_copy(zrow_hbm, zrow_v)
        pltpu.sync_copy(zvec_hbm.at[pl.ds(0, per_node)], zvec_v)
        row0 = sid * per_node

        @pl.loop(0, per_node // zr)
        def _(j):
            pltpu.sync_copy(zrow_v, acc_agg.at[pl.ds(row0 + j * zr, zr)])

        pltpu.sync_copy(zvec_v, acc_s.at[pl.ds(row0, per_node)])
        plsc.subcore_barrier()

        base0 = wid * per_tile

        @pl.loop(0, n_chunks)
        def _(ci):
            base = base0 + ci * CHUNK
            pltpu.sync_copy(src_hbm.at[pl.ds(base, CHUNK)], src_v)
            pltpu.sync_copy(dst_hbm.at[pl.ds(base, CHUNK)], dst_v)
            pltpu.async_copy(hs_hbm.at[src_v], rows_v, sem_r).wait()
            pltpu.sync_copy(rows_v, acc_agg.at[dst_v], add=True)
            pltpu.async_copy(dinv_hbm.at[dst_v], dval_v, sem_d).wait()
            pltpu.sync_copy(dval_v, acc_s.at[src_v], add=True)

        plsc.subcore_barrier()
        sl = pl.ds(row0, per_node)
        pltpu.sync_copy(acc_agg.at[sl], agg_hbm.at[cid, sl])
        pltpu.sync_copy(acc_s.at[sl], s_hbm.at[cid, sl])

    return k(hs, dinv_flat, src_p, dst_p, zrow, zvec)


def _tc_prepare(degp, x_p, W1, n, n_pad, blk):
    nb = n_pad // blk
    in_dim = x_p.shape[1]
    hid = W1.shape[1]

    def body(degp_ref, x_ref, w1_ref, dinv_ref, hs_ref):
        i = pl.program_id(0)
        d = degp_ref[0] + degp_ref[1] + 1.0
        rows = lax.broadcasted_iota(jnp.int32, (blk, 1), 0) + i * blk
        dinv = jnp.where(rows < n, lax.rsqrt(d), 0.0)
        dinv_ref[...] = dinv
        h = jnp.dot(x_ref[...], w1_ref[...], preferred_element_type=jnp.float32)
        hs_ref[...] = h * dinv

    return pl.pallas_call(
        body,
        grid=(nb,),
        in_specs=[
            pl.BlockSpec((NC, blk, 1), lambda i: (0, i, 0)),
            pl.BlockSpec((blk, in_dim), lambda i: (i, 0)),
            pl.BlockSpec((in_dim, hid), lambda i: (0, 0)),
        ],
        out_specs=[
            pl.BlockSpec((blk, 1), lambda i: (i, 0)),
            pl.BlockSpec((blk, hid), lambda i: (i, 0)),
        ],
        out_shape=[
            jax.ShapeDtypeStruct((n_pad, 1), jnp.float32),
            jax.ShapeDtypeStruct((n_pad, hid), jnp.float32),
        ],
    )(degp, x_p, W1)


def _tc_final(aggp, hs, dinv, sp, b1, W2, b2, n, n_pad, blk):
    nb = n_pad // blk
    hid = hs.shape[1]
    out_dim = W2.shape[1]

    def body(aggp_ref, hs_ref, dinv_ref, sp_ref, b1_ref, w2_ref, b2_ref,
             out_ref, acc_ref):
        i = pl.program_id(0)
        agg = aggp_ref[0] + aggp_ref[1]
        dinv = dinv_ref[...]
        s = sp_ref[0] + sp_ref[1]
        h1 = jnp.maximum(dinv * (agg + hs_ref[...]) + b1_ref[...], 0.0)
        c = dinv * (s + dinv)
        part = jnp.sum(c * h1, axis=0, keepdims=True)

        @pl.when(i == 0)
        def _():
            acc_ref[...] = jnp.zeros_like(acc_ref)

        acc_ref[...] += part

        @pl.when(i == nb - 1)
        def _():
            v = acc_ref[...] * (1.0 / n)
            out_ref[...] = (
                jnp.dot(v, w2_ref[...], preferred_element_type=jnp.float32)
                + b2_ref[...]
            )

    return pl.pallas_call(
        body,
        grid=(nb,),
        in_specs=[
            pl.BlockSpec((NC, blk, hid), lambda i: (0, i, 0)),
            pl.BlockSpec((blk, hid), lambda i: (i, 0)),
            pl.BlockSpec((blk, 1), lambda i: (i, 0)),
            pl.BlockSpec((NC, blk, 1), lambda i: (0, i, 0)),
            pl.BlockSpec((1, hid), lambda i: (0, 0)),
            pl.BlockSpec((hid, out_dim), lambda i: (0, 0)),
            pl.BlockSpec((1, out_dim), lambda i: (0, 0)),
        ],
        out_specs=pl.BlockSpec((1, out_dim), lambda i: (0, 0)),
        out_shape=jax.ShapeDtypeStruct((1, out_dim), jnp.float32),
        scratch_shapes=[pltpu.VMEM((1, hid), jnp.float32)],
    )(aggp, hs, dinv, sp, b1, W2, b2)


def kernel(x, edge_index, W1, b1, W2, b2):
    n, in_dim = x.shape
    hid = W1.shape[1]
    e = edge_index.shape[1]

    n_pad = -(-n // 2048) * 2048
    e_pad = -(-e // (NW * CHUNK)) * (NW * CHUNK)
    blk = 512
    per_node = n_pad // NS

    src = edge_index[0].astype(jnp.int32)
    dst = edge_index[1].astype(jnp.int32)
    pad_e = e_pad - e
    src_p = jnp.concatenate([src, jnp.zeros((pad_e,), jnp.int32)])
    dst_p = jnp.concatenate(
        [dst, n + (jnp.arange(pad_e, dtype=jnp.int32) % (n_pad - n))])

    x_p = jnp.concatenate([x, jnp.zeros((n_pad - n, in_dim), x.dtype)])

    zvec = jnp.zeros((per_node,), jnp.float32)
    ones = jnp.ones((CHUNK,), jnp.float32)
    zrow = jnp.zeros((CHUNK, hid), jnp.float32)

    degp = _sc_degree(dst_p, zvec, ones, n_pad, e_pad)
    dinv, hs = _tc_prepare(
        degp.reshape(NC, n_pad, 1), x_p, W1, n, n_pad, blk)
    aggp, sp = _sc_edge_aggregate(
        hs, dinv.reshape(n_pad), src_p, dst_p, zrow, zvec, n_pad, e_pad, hid)
    out = _tc_final(
        aggp, hs, dinv, sp.reshape(NC, n_pad, 1),
        b1.reshape(1, hid), W2, b2.reshape(1, -1), n, n_pad, blk)
    return out


# speedup vs baseline: 14.8541x; 1.3950x over previous
import functools

import jax
import jax.numpy as jnp
from jax import lax
from jax.experimental import pallas as pl
from jax.experimental.pallas import tpu as pltpu
from jax.experimental.pallas import tpu_sc as plsc

NC = 2
NS = 16
NW = NC * NS
CHUNK = 128
GB = 8
RB = 2

_MESH = plsc.VectorSubcoreMesh(core_axis_name="c", subcore_axis_name="s")


def _sc_degree(dst2d, n_pad, e_pad):
    nd = e_pad // (NW * CHUNK)
    per_node = n_pad // NS

    @functools.partial(
        pl.kernel,
        mesh=_MESH,
        out_type=jax.ShapeDtypeStruct((NC, n_pad), jnp.float32),
        scratch_types=[
            pltpu.VMEM((nd, CHUNK), jnp.int32),
            pltpu.VMEM((CHUNK,), jnp.float32),
            pltpu.VMEM((per_node,), jnp.float32),
            pltpu.VMEM_SHARED((n_pad,), jnp.float32),
            pltpu.SemaphoreType.DMA,
            pltpu.SemaphoreType.DMA,
        ],
    )
    def k(dst_hbm, out_hbm, idx_v, ones_v, z_v, acc_sh, sem_i, sem_w):
        cid = lax.axis_index("c")
        sid = lax.axis_index("s")
        wid = sid * NC + cid
        cp = pltpu.async_copy(dst_hbm.at[pl.ds(wid * nd, nd)], idx_v, sem_i)

        @pl.loop(0, CHUNK, step=16)
        def _(i):
            ones_v[pl.ds(i, 16)] = jnp.ones((16,), jnp.float32)

        @pl.loop(0, per_node, step=16)
        def _(i):
            z_v[pl.ds(i, 16)] = jnp.zeros((16,), jnp.float32)

        pltpu.sync_copy(z_v, acc_sh.at[pl.ds(sid * per_node, per_node)])
        cp.wait()
        plsc.subcore_barrier()

        @pl.loop(0, nd // GB)
        def _(g):
            for b in range(GB):
                pltpu.async_copy(
                    ones_v, acc_sh.at[idx_v.at[g * GB + b]], sem_w, add=True)
            for b in range(GB):
                pltpu.make_async_copy(
                    ones_v, acc_sh.at[idx_v.at[0]], sem_w).wait()

        plsc.subcore_barrier()
        sl = pl.ds(sid * per_node, per_node)
        pltpu.sync_copy(acc_sh.at[sl], out_hbm.at[cid, sl])

    return k(dst2d)


def _sc_edge_aggregate(hs, dinv_flat, src2d, dst2d, n_pad, e_pad, hid):
    nd = e_pad // (NW * CHUNK)
    ngr = nd // GB
    per_node = n_pad // NS
    zc = per_node // CHUNK

    @functools.partial(
        pl.kernel,
        mesh=_MESH,
        out_type=(
            jax.ShapeDtypeStruct((NC, n_pad, hid), jnp.float32),
            jax.ShapeDtypeStruct((NC, n_pad), jnp.float32),
        ),
        scratch_types=[
            pltpu.VMEM((nd, CHUNK), jnp.int32),
            pltpu.VMEM((nd, CHUNK), jnp.int32),
            pltpu.VMEM((RB, CHUNK, hid), jnp.float32),
            pltpu.VMEM((GB, CHUNK), jnp.float32),
            pltpu.VMEM((per_node,), jnp.float32),
            pltpu.VMEM_SHARED((n_pad, hid), jnp.float32),
            pltpu.VMEM_SHARED((n_pad,), jnp.float32),
            pltpu.SemaphoreType.DMA((2,)),
            pltpu.SemaphoreType.DMA((RB,)),
            pltpu.SemaphoreType.DMA((RB,)),
            pltpu.SemaphoreType.DMA((GB,)),
            pltpu.SemaphoreType.DMA((GB,)),
        ],
    )
    def k(hs_hbm, dinv_hbm, src_hbm, dst_hbm, agg_hbm, s_hbm,
          src_v, dst_v, rows_v, dval_v, z_v, acc_agg, acc_s,
          sem_i, sem_g, sem_s, sem_dg, sem_ss):
        cid = lax.axis_index("c")
        sid = lax.axis_index("s")
        wid = sid * NC + cid
        row0 = sid * per_node
        c0 = wid * nd

        cpa = pltpu.async_copy(src_hbm.at[pl.ds(c0, nd)], src_v, sem_i.at[0])
        cpb = pltpu.async_copy(dst_hbm.at[pl.ds(c0, nd)], dst_v, sem_i.at[1])

        @pl.loop(0, CHUNK)
        def _(r):
            @pl.loop(0, hid, step=16)
            def _(cc):
                rows_v[0, r, pl.ds(cc, 16)] = jnp.zeros((16,), jnp.float32)

        @pl.loop(0, per_node, step=16)
        def _(i):
            z_v[pl.ds(i, 16)] = jnp.zeros((16,), jnp.float32)

        for j in range(zc):
            pltpu.sync_copy(
                rows_v.at[0], acc_agg.at[pl.ds(row0 + j * CHUNK, CHUNK)])
        pltpu.sync_copy(z_v, acc_s.at[pl.ds(row0, per_node)])
        cpa.wait()
        cpb.wait()
        plsc.subcore_barrier()

        pltpu.async_copy(hs_hbm.at[src_v.at[0]], rows_v.at[0], sem_g.at[0])
        for j in range(4):
            pltpu.async_copy(dinv_hbm.at[dst_v.at[j]], dval_v.at[j],
                             sem_dg.at[j])

        @pl.loop(0, ngr)
        def _(g):
            for b in range(GB):
                ck = g * GB + b
                br = b % RB
                b1 = (b + 1) % RB
                b4 = (b + 4) % GB

                def _rows_ahead():
                    pltpu.make_async_copy(
                        rows_v.at[b1], acc_agg.at[dst_v.at[0]],
                        sem_s.at[b1]).wait()
                    pltpu.async_copy(hs_hbm.at[src_v.at[ck + 1]],
                                     rows_v.at[b1], sem_g.at[b1])

                if b == 0:
                    @pl.when(g >= 1)
                    def _():
                        pltpu.make_async_copy(
                            rows_v.at[b1], acc_agg.at[dst_v.at[0]],
                            sem_s.at[b1]).wait()
                    pltpu.async_copy(hs_hbm.at[src_v.at[ck + 1]],
                                     rows_v.at[b1], sem_g.at[b1])
                elif b < GB - 1:
                    _rows_ahead()
                else:
                    @pl.when(g < ngr - 1)
                    def _():
                        _rows_ahead()

                pltpu.make_async_copy(
                    hs_hbm.at[src_v.at[ck]], rows_v.at[br],
                    sem_g.at[br]).wait()
                pltpu.async_copy(rows_v.at[br], acc_agg.at[dst_v.at[ck]],
                                 sem_s.at[br], add=True)

                def _scal_ahead():
                    pltpu.make_async_copy(
                        dval_v.at[b4], acc_s.at[src_v.at[0]],
                        sem_ss.at[b4]).wait()
                    pltpu.async_copy(dinv_hbm.at[dst_v.at[ck + 4]],
                                     dval_v.at[b4], sem_dg.at[b4])

                if b < 4:
                    @pl.when(g >= 1)
                    def _():
                        pltpu.make_async_copy(
                            dval_v.at[b4], acc_s.at[src_v.at[0]],
                            sem_ss.at[b4]).wait()
                    pltpu.async_copy(dinv_hbm.at[dst_v.at[ck + 4]],
                                     dval_v.at[b4], sem_dg.at[b4])
                else:
                    @pl.when(g < ngr - 1)
                    def _():
                        _scal_ahead()

                pltpu.make_async_copy(
                    dinv_hbm.at[dst_v.at[ck]], dval_v.at[b],
                    sem_dg.at[b]).wait()
                pltpu.async_copy(dval_v.at[b], acc_s.at[src_v.at[ck]],
                                 sem_ss.at[b], add=True)

        for b in range(RB):
            pltpu.make_async_copy(
                rows_v.at[b], acc_agg.at[dst_v.at[0]], sem_s.at[b]).wait()
        for b in range(GB):
            pltpu.make_async_copy(
                dval_v.at[b], acc_s.at[src_v.at[0]], sem_ss.at[b]).wait()

        plsc.subcore_barrier()
        sl = pl.ds(row0, per_node)
        pltpu.sync_copy(acc_agg.at[sl], agg_hbm.at[cid, sl])
        pltpu.sync_copy(acc_s.at[sl], s_hbm.at[cid, sl])

    return k(hs, dinv_flat, src2d, dst2d)


def _tc_matmul(x_p, W1, n_pad, blk):
    nb = n_pad // blk
    in_dim = x_p.shape[1]
    hid = W1.shape[1]

    def body(x_ref, w1_ref, h_ref):
        h_ref[...] = jnp.dot(x_ref[...], w1_ref[...],
                             preferred_element_type=jnp.float32)

    return pl.pallas_call(
        body,
        grid=(nb,),
        in_specs=[
            pl.BlockSpec((blk, in_dim), lambda i: (i, 0)),
            pl.BlockSpec((in_dim, hid), lambda i: (0, 0)),
        ],
        out_specs=pl.BlockSpec((blk, hid), lambda i: (i, 0)),
        out_shape=jax.ShapeDtypeStruct((n_pad, hid), jnp.float32),
    )(x_p, W1)


def _tc_scale(degp, h, n, n_pad, blk):
    nb = n_pad // blk
    hid = h.shape[1]

    def body(degp_ref, h_ref, dinv_ref, hs_ref):
        i = pl.program_id(0)
        d = degp_ref[0] + degp_ref[1] + 1.0
        rows = lax.broadcasted_iota(jnp.int32, (blk, 1), 0) + i * blk
        dinv = jnp.where(rows < n, lax.rsqrt(d), 0.0)
        dinv_ref[...] = dinv
        hs_ref[...] = h_ref[...] * dinv

    return pl.pallas_call(
        body,
        grid=(nb,),
        in_specs=[
            pl.BlockSpec((NC, blk, 1), lambda i: (0, i, 0)),
            pl.BlockSpec((blk, hid), lambda i: (i, 0)),
        ],
        out_specs=[
            pl.BlockSpec((blk, 1), lambda i: (i, 0)),
            pl.BlockSpec((blk, hid), lambda i: (i, 0)),
        ],
        out_shape=[
            jax.ShapeDtypeStruct((n_pad, 1), jnp.float32),
            jax.ShapeDtypeStruct((n_pad, hid), jnp.float32),
        ],
    )(degp, h)


def _tc_final(aggp, hs, dinv, sp, b1, W2, b2, n, n_pad, blk):
    nb = n_pad // blk
    hid = hs.shape[1]
    out_dim = W2.shape[1]

    def body(aggp_ref, hs_ref, dinv_ref, sp_ref, b1_ref, w2_ref, b2_ref,
             out_ref, acc_ref):
        i = pl.program_id(0)
        agg = aggp_ref[0] + aggp_ref[1]
        dinv = dinv_ref[...]
        s = sp_ref[0] + sp_ref[1]
        h1 = jnp.maximum(dinv * (agg + hs_ref[...]) + b1_ref[...], 0.0)
        c = dinv * (s + dinv)
        part = jnp.sum(c * h1, axis=0, keepdims=True)

        @pl.when(i == 0)
        def _():
            acc_ref[...] = jnp.zeros_like(acc_ref)

        acc_ref[...] += part

        @pl.when(i == nb - 1)
        def _():
            v = acc_ref[...] * (1.0 / n)
            out_ref[...] = (
                jnp.dot(v, w2_ref[...], preferred_element_type=jnp.float32)
                + b2_ref[...]
            )

    return pl.pallas_call(
        body,
        grid=(nb,),
        in_specs=[
            pl.BlockSpec((NC, blk, hid), lambda i: (0, i, 0)),
            pl.BlockSpec((blk, hid), lambda i: (i, 0)),
            pl.BlockSpec((blk, 1), lambda i: (i, 0)),
            pl.BlockSpec((NC, blk, 1), lambda i: (0, i, 0)),
            pl.BlockSpec((1, hid), lambda i: (0, 0)),
            pl.BlockSpec((hid, out_dim), lambda i: (0, 0)),
            pl.BlockSpec((1, out_dim), lambda i: (0, 0)),
        ],
        out_specs=pl.BlockSpec((1, out_dim), lambda i: (0, 0)),
        out_shape=jax.ShapeDtypeStruct((1, out_dim), jnp.float32),
        scratch_shapes=[pltpu.VMEM((1, hid), jnp.float32)],
    )(aggp, hs, dinv, sp, b1, W2, b2)


def kernel(x, edge_index, W1, b1, W2, b2):
    n, in_dim = x.shape
    hid = W1.shape[1]
    e = edge_index.shape[1]

    n_pad = -(-n // 2048) * 2048
    egrain = NW * CHUNK * GB
    e_pad = -(-e // egrain) * egrain
    blk = 512

    src = edge_index[0].astype(jnp.int32)
    dst = edge_index[1].astype(jnp.int32)
    pad_e = e_pad - e
    src_p = jnp.concatenate([src, jnp.zeros((pad_e,), jnp.int32)])
    dst_p = jnp.concatenate(
        [dst, n + (jnp.arange(pad_e, dtype=jnp.int32) % (n_pad - n))])
    src2d = src_p.reshape(e_pad // CHUNK, CHUNK)
    dst2d = dst_p.reshape(e_pad // CHUNK, CHUNK)

    x_p = jnp.concatenate([x, jnp.zeros((n_pad - n, in_dim), x.dtype)])

    degp = _sc_degree(dst2d, n_pad, e_pad)
    h = _tc_matmul(x_p, W1, n_pad, blk)
    dinv, hs = _tc_scale(degp.reshape(NC, n_pad, 1), h, n, n_pad, blk)
    aggp, sp = _sc_edge_aggregate(
        hs, dinv.reshape(n_pad), src2d, dst2d, n_pad, e_pad, hid)
    out = _tc_final(
        aggp, hs, dinv, sp.reshape(NC, n_pad, 1),
        b1.reshape(1, hid), W2, b2.reshape(1, -1), n, n_pad, blk)
    return out


# speedup vs baseline: 30.2834x; 2.0387x over previous
import functools

import jax
import jax.numpy as jnp
from jax import lax
from jax.experimental import pallas as pl
from jax.experimental.pallas import tpu as pltpu
from jax.experimental.pallas import tpu_sc as plsc

NC = 2
NS = 16
NW = NC * NS
CHUNK = 128
GB = 8
RB = 2

_MESH = plsc.VectorSubcoreMesh(core_axis_name="c", subcore_axis_name="s")


def _sc_degree(dst2d, n_pad, e_pad):
    nd = e_pad // (NW * CHUNK)
    per_node = n_pad // NS

    @functools.partial(
        pl.kernel,
        mesh=_MESH,
        out_type=jax.ShapeDtypeStruct((NC, n_pad), jnp.float32),
        scratch_types=[
            pltpu.VMEM((nd, CHUNK), jnp.int32),
            pltpu.VMEM((CHUNK,), jnp.float32),
            pltpu.VMEM((per_node,), jnp.float32),
            pltpu.VMEM_SHARED((n_pad,), jnp.float32),
            pltpu.SemaphoreType.DMA,
            pltpu.SemaphoreType.DMA,
        ],
    )
    def k(dst_hbm, out_hbm, idx_v, ones_v, z_v, acc_sh, sem_i, sem_w):
        cid = lax.axis_index("c")
        sid = lax.axis_index("s")
        wid = sid * NC + cid
        cp = pltpu.async_copy(dst_hbm.at[pl.ds(wid * nd, nd)], idx_v, sem_i)

        @pl.loop(0, CHUNK, step=16)
        def _(i):
            ones_v[pl.ds(i, 16)] = jnp.ones((16,), jnp.float32)

        @pl.loop(0, per_node, step=16)
        def _(i):
            z_v[pl.ds(i, 16)] = jnp.zeros((16,), jnp.float32)

        pltpu.sync_copy(z_v, acc_sh.at[pl.ds(sid * per_node, per_node)])
        cp.wait()
        plsc.subcore_barrier()

        @pl.loop(0, nd // GB)
        def _(g):
            for b in range(GB):
                pltpu.async_copy(
                    ones_v, acc_sh.at[idx_v.at[g * GB + b]], sem_w, add=True)
            for b in range(GB):
                pltpu.make_async_copy(
                    ones_v, acc_sh.at[idx_v.at[0]], sem_w).wait()

        plsc.subcore_barrier()
        sl = pl.ds(sid * per_node, per_node)
        pltpu.sync_copy(acc_sh.at[sl], out_hbm.at[cid, sl])

    return k(dst2d)


def _sc_edge_aggregate(hs, dinv_flat, src2d, dst2d, n_pad, e_pad, hid):
    nd = e_pad // (NW * CHUNK)
    ngr = nd // GB
    per_node = n_pad // NS
    zc = per_node // CHUNK

    @functools.partial(
        pl.kernel,
        mesh=_MESH,
        out_type=(
            jax.ShapeDtypeStruct((NC, n_pad, hid), jnp.float32),
            jax.ShapeDtypeStruct((NC, n_pad), jnp.float32),
        ),
        scratch_types=[
            pltpu.VMEM((nd, CHUNK), jnp.int32),
            pltpu.VMEM((nd, CHUNK), jnp.int32),
            pltpu.VMEM((RB, CHUNK, hid), jnp.float32),
            pltpu.VMEM((GB, CHUNK), jnp.float32),
            pltpu.VMEM((per_node,), jnp.float32),
            pltpu.VMEM_SHARED((n_pad, hid), jnp.float32),
            pltpu.VMEM_SHARED((n_pad,), jnp.float32),
            pltpu.SemaphoreType.DMA((2,)),
            pltpu.SemaphoreType.DMA((RB,)),
            pltpu.SemaphoreType.DMA((RB,)),
            pltpu.SemaphoreType.DMA((GB,)),
            pltpu.SemaphoreType.DMA((GB,)),
        ],
    )
    def k(hs_hbm, dinv_hbm, src_hbm, dst_hbm, agg_hbm, s_hbm,
          src_v, dst_v, rows_v, dval_v, z_v, acc_agg, acc_s,
          sem_i, sem_g, sem_s, sem_dg, sem_ss):
        cid = lax.axis_index("c")
        sid = lax.axis_index("s")
        wid = sid * NC + cid
        row0 = sid * per_node
        c0 = wid * nd

        cpa = pltpu.async_copy(src_hbm.at[pl.ds(c0, nd)], src_v, sem_i.at[0])
        cpb = pltpu.async_copy(dst_hbm.at[pl.ds(c0, nd)], dst_v, sem_i.at[1])

        @pl.loop(0, CHUNK)
        def _(r):
            @pl.loop(0, hid, step=16)
            def _(cc):
                rows_v[0, r, pl.ds(cc, 16)] = jnp.zeros((16,), jnp.float32)

        @pl.loop(0, per_node, step=16)
        def _(i):
            z_v[pl.ds(i, 16)] = jnp.zeros((16,), jnp.float32)

        for j in range(zc):
            pltpu.sync_copy(
                rows_v.at[0], acc_agg.at[pl.ds(row0 + j * CHUNK, CHUNK)])
        pltpu.sync_copy(z_v, acc_s.at[pl.ds(row0, per_node)])
        cpa.wait()
        cpb.wait()
        plsc.subcore_barrier()

        pltpu.async_copy(hs_hbm.at[src_v.at[0]], rows_v.at[0], sem_g.at[0])
        for j in range(4):
            pltpu.async_copy(dinv_hbm.at[dst_v.at[j]], dval_v.at[j],
                             sem_dg.at[j])

        @pl.loop(0, ngr)
        def _(g):
            for b in range(GB):
                ck = g * GB + b
                br = b % RB
                b1 = (b + 1) % RB
                b4 = (b + 4) % GB

                def _rows_ahead():
                    pltpu.make_async_copy(
                        rows_v.at[b1], acc_agg.at[dst_v.at[0]],
                        sem_s.at[b1]).wait()
                    pltpu.async_copy(hs_hbm.at[src_v.at[ck + 1]],
                                     rows_v.at[b1], sem_g.at[b1])

                if b == 0:
                    @pl.when(g >= 1)
                    def _():
                        pltpu.make_async_copy(
                            rows_v.at[b1], acc_agg.at[dst_v.at[0]],
                            sem_s.at[b1]).wait()
                    pltpu.async_copy(hs_hbm.at[src_v.at[ck + 1]],
                                     rows_v.at[b1], sem_g.at[b1])
                elif b < GB - 1:
                    _rows_ahead()
                else:
                    @pl.when(g < ngr - 1)
                    def _():
                        _rows_ahead()

                pltpu.make_async_copy(
                    hs_hbm.at[src_v.at[ck]], rows_v.at[br],
                    sem_g.at[br]).wait()
                pltpu.async_copy(rows_v.at[br], acc_agg.at[dst_v.at[ck]],
                                 sem_s.at[br], add=True)

                def _scal_ahead():
                    pltpu.make_async_copy(
                        dval_v.at[b4], acc_s.at[src_v.at[0]],
                        sem_ss.at[b4]).wait()
                    pltpu.async_copy(dinv_hbm.at[dst_v.at[ck + 4]],
                                     dval_v.at[b4], sem_dg.at[b4])

                if b < 4:
                    @pl.when(g >= 1)
                    def _():
                        pltpu.make_async_copy(
                            dval_v.at[b4], acc_s.at[src_v.at[0]],
                            sem_ss.at[b4]).wait()
                    pltpu.async_copy(dinv_hbm.at[dst_v.at[ck + 4]],
                                     dval_v.at[b4], sem_dg.at[b4])
                else:
                    @pl.when(g < ngr - 1)
                    def _():
                        _scal_ahead()

                pltpu.make_async_copy(
                    dinv_hbm.at[dst_v.at[ck]], dval_v.at[b],
                    sem_dg.at[b]).wait()
                pltpu.async_copy(dval_v.at[b], acc_s.at[src_v.at[ck]],
                                 sem_ss.at[b], add=True)

        for b in range(RB):
            pltpu.make_async_copy(
                rows_v.at[b], acc_agg.at[dst_v.at[0]], sem_s.at[b]).wait()
        for b in range(GB):
            pltpu.make_async_copy(
                dval_v.at[b], acc_s.at[src_v.at[0]], sem_ss.at[b]).wait()

        plsc.subcore_barrier()
        sl = pl.ds(row0, per_node)
        pltpu.sync_copy(acc_agg.at[sl], agg_hbm.at[cid, sl])
        pltpu.sync_copy(acc_s.at[sl], s_hbm.at[cid, sl])

    return k(hs, dinv_flat, src2d, dst2d)


def _tc_matmul(x_p, W1, n_pad, blk):
    nb = n_pad // blk
    in_dim = x_p.shape[1]
    hid = W1.shape[1]

    def body(x_ref, w1_ref, h_ref):
        h_ref[...] = jnp.dot(x_ref[...], w1_ref[...],
                             preferred_element_type=jnp.float32)

    return pl.pallas_call(
        body,
        grid=(nb,),
        in_specs=[
            pl.BlockSpec((blk, in_dim), lambda i: (i, 0)),
            pl.BlockSpec((in_dim, hid), lambda i: (0, 0)),
        ],
        out_specs=pl.BlockSpec((blk, hid), lambda i: (i, 0)),
        out_shape=jax.ShapeDtypeStruct((n_pad, hid), jnp.float32),
    )(x_p, W1)


def _tc_scale(degp, h, n, n_pad, blk):
    nb = n_pad // blk
    hid = h.shape[1]

    def body(degp_ref, h_ref, dinv_ref, hs_ref):
        i = pl.program_id(0)
        d = degp_ref[0] + degp_ref[1] + 1.0
        rows = lax.broadcasted_iota(jnp.int32, (blk, 1), 0) + i * blk
        dinv = jnp.where(rows < n, lax.rsqrt(d), 0.0)
        dinv_ref[...] = dinv
        hs_ref[...] = h_ref[...] * dinv

    return pl.pallas_call(
        body,
        grid=(nb,),
        in_specs=[
            pl.BlockSpec((NC, blk, 1), lambda i: (0, i, 0)),
            pl.BlockSpec((blk, hid), lambda i: (i, 0)),
        ],
        out_specs=[
            pl.BlockSpec((blk, 1), lambda i: (i, 0)),
            pl.BlockSpec((blk, hid), lambda i: (i, 0)),
        ],
        out_shape=[
            jax.ShapeDtypeStruct((n_pad, 1), jnp.float32),
            jax.ShapeDtypeStruct((n_pad, hid), jnp.float32),
        ],
    )(degp, h)


def _tc_final(aggp, hs, dinv, sp, b1, W2, b2, n, n_pad, blk):
    nb = n_pad // blk
    hid = hs.shape[1]
    out_dim = W2.shape[1]

    def body(aggp_ref, hs_ref, dinv_ref, sp_ref, b1_ref, w2_ref, b2_ref,
             out_ref, acc_ref):
        i = pl.program_id(0)
        agg = aggp_ref[0] + aggp_ref[1]
        dinv = dinv_ref[...]
        s = sp_ref[0] + sp_ref[1]
        h1 = jnp.maximum(dinv * (agg + hs_ref[...]) + b1_ref[...], 0.0)
        c = dinv * (s + dinv)
        part = jnp.sum(c * h1, axis=0, keepdims=True)

        @pl.when(i == 0)
        def _():
            acc_ref[...] = jnp.zeros_like(acc_ref)

        acc_ref[...] += part

        @pl.when(i == nb - 1)
        def _():
            v = acc_ref[...] * (1.0 / n)
            out_ref[...] = (
                jnp.dot(v, w2_ref[...], preferred_element_type=jnp.float32)
                + b2_ref[...]
            )

    return pl.pallas_call(
        body,
        grid=(nb,),
        in_specs=[
            pl.BlockSpec((NC, blk, hid), lambda i: (0, i, 0)),
            pl.BlockSpec((blk, hid), lambda i: (i, 0)),
            pl.BlockSpec((blk, 1), lambda i: (i, 0)),
            pl.BlockSpec((NC, blk, 1), lambda i: (0, i, 0)),
            pl.BlockSpec((1, hid), lambda i: (0, 0)),
            pl.BlockSpec((hid, out_dim), lambda i: (0, 0)),
            pl.BlockSpec((1, out_dim), lambda i: (0, 0)),
        ],
        out_specs=pl.BlockSpec((1, out_dim), lambda i: (0, 0)),
        out_shape=jax.ShapeDtypeStruct((1, out_dim), jnp.float32),
        scratch_shapes=[pltpu.VMEM((1, hid), jnp.float32)],
    )(aggp, hs, dinv, sp, b1, W2, b2)


def kernel(x, edge_index, W1, b1, W2, b2):
    n, in_dim = x.shape
    hid = W1.shape[1]
    e = edge_index.shape[1]

    n_pad = -(-n // 2048) * 2048
    egrain = NW * CHUNK * GB
    e_pad = -(-e // egrain) * egrain
    blk = 512

    src = edge_index[0].astype(jnp.int32)
    dst = edge_index[1].astype(jnp.int32)
    pad_e = e_pad - e
    pad_idx = n + (jnp.arange(pad_e, dtype=jnp.int32) % (n_pad - n))
    src_p = jnp.concatenate([src, pad_idx])
    dst_p = jnp.concatenate([dst, pad_idx])
    src2d = src_p.reshape(e_pad // CHUNK, CHUNK)
    dst2d = dst_p.reshape(e_pad // CHUNK, CHUNK)

    x_p = jnp.concatenate([x, jnp.zeros((n_pad - n, in_dim), x.dtype)])

    degp = _sc_degree(dst2d, n_pad, e_pad)
    h = _tc_matmul(x_p, W1, n_pad, blk)
    dinv, hs = _tc_scale(degp.reshape(NC, n_pad, 1), h, n, n_pad, blk)
    aggp, sp = _sc_edge_aggregate(
        hs, dinv.reshape(n_pad), src2d, dst2d, n_pad, e_pad, hid)
    out = _tc_final(
        aggp, hs, dinv, sp.reshape(NC, n_pad, 1),
        b1.reshape(1, hid), W2, b2.reshape(1, -1), n, n_pad, blk)
    return out


# speedup vs baseline: 38.9962x; 1.2877x over previous
import functools

import jax
import jax.numpy as jnp
import numpy as np
from jax import lax
from jax.experimental import pallas as pl
from jax.experimental.pallas import tpu as pltpu
from jax.experimental.pallas import tpu_sc as plsc

NC = 2
NS = 16
NW = NC * NS
CHUNK = 128
GB = 8
RB = 2

_MESH = plsc.VectorSubcoreMesh(core_axis_name="c", subcore_axis_name="s")


def _sc_degree(dst2d, n_pad, e_pad):
    nd = e_pad // (NW * CHUNK)
    per_node = n_pad // NS

    @functools.partial(
        pl.kernel,
        mesh=_MESH,
        out_type=jax.ShapeDtypeStruct((NC, n_pad), jnp.float32),
        scratch_types=[
            pltpu.VMEM((nd, CHUNK), jnp.int32),
            pltpu.VMEM((CHUNK,), jnp.float32),
            pltpu.VMEM((per_node,), jnp.float32),
            pltpu.VMEM_SHARED((n_pad,), jnp.float32),
            pltpu.SemaphoreType.DMA,
            pltpu.SemaphoreType.DMA,
        ],
    )
    def k(dst_hbm, out_hbm, idx_v, ones_v, z_v, acc_sh, sem_i, sem_w):
        cid = lax.axis_index("c")
        sid = lax.axis_index("s")
        wid = sid * NC + cid
        cp = pltpu.async_copy(dst_hbm.at[pl.ds(wid * nd, nd)], idx_v, sem_i)

        @pl.loop(0, CHUNK, step=16)
        def _(i):
            ones_v[pl.ds(i, 16)] = jnp.ones((16,), jnp.float32)

        @pl.loop(0, per_node, step=16)
        def _(i):
            z_v[pl.ds(i, 16)] = jnp.zeros((16,), jnp.float32)

        pltpu.sync_copy(z_v, acc_sh.at[pl.ds(sid * per_node, per_node)])
        cp.wait()
        plsc.subcore_barrier()

        @pl.loop(0, nd // GB)
        def _(g):
            for b in range(GB):
                pltpu.async_copy(
                    ones_v, acc_sh.at[idx_v.at[g * GB + b]], sem_w, add=True)
            for b in range(GB):
                pltpu.make_async_copy(
                    ones_v, acc_sh.at[idx_v.at[0]], sem_w).wait()

        plsc.subcore_barrier()
        sl = pl.ds(sid * per_node, per_node)
        pltpu.sync_copy(acc_sh.at[sl], out_hbm.at[cid, sl])

    return k(dst2d)


def _sc_edge_aggregate(hs, dinv_flat, src2d, dst2d, n_pad, e_pad, hid):
    nd = e_pad // (NW * CHUNK)
    ngr = nd // GB
    per_node = n_pad // NS
    zc = per_node // CHUNK

    @functools.partial(
        pl.kernel,
        mesh=_MESH,
        out_type=(
            jax.ShapeDtypeStruct((NC, n_pad, hid), jnp.float32),
            jax.ShapeDtypeStruct((NC, n_pad), jnp.float32),
        ),
        scratch_types=[
            pltpu.VMEM((nd, CHUNK), jnp.int32),
            pltpu.VMEM((nd, CHUNK), jnp.int32),
            pltpu.VMEM((RB, CHUNK, hid), jnp.float32),
            pltpu.VMEM((GB, CHUNK), jnp.float32),
            pltpu.VMEM((per_node,), jnp.float32),
            pltpu.VMEM_SHARED((n_pad, hid), jnp.float32),
            pltpu.VMEM_SHARED((n_pad,), jnp.float32),
            pltpu.SemaphoreType.DMA((2,)),
            pltpu.SemaphoreType.DMA((RB,)),
            pltpu.SemaphoreType.DMA((RB,)),
            pltpu.SemaphoreType.DMA((GB,)),
            pltpu.SemaphoreType.DMA((GB,)),
        ],
    )
    def k(hs_hbm, dinv_hbm, src_hbm, dst_hbm, agg_hbm, s_hbm,
          src_v, dst_v, rows_v, dval_v, z_v, acc_agg, acc_s,
          sem_i, sem_g, sem_s, sem_dg, sem_ss):
        cid = lax.axis_index("c")
        sid = lax.axis_index("s")
        wid = sid * NC + cid
        row0 = sid * per_node
        c0 = wid * nd

        cpa = pltpu.async_copy(src_hbm.at[pl.ds(c0, nd)], src_v, sem_i.at[0])
        cpb = pltpu.async_copy(dst_hbm.at[pl.ds(c0, nd)], dst_v, sem_i.at[1])

        @pl.loop(0, CHUNK)
        def _(r):
            @pl.loop(0, hid, step=16)
            def _(cc):
                rows_v[0, r, pl.ds(cc, 16)] = jnp.zeros((16,), jnp.float32)

        @pl.loop(0, per_node, step=16)
        def _(i):
            z_v[pl.ds(i, 16)] = jnp.zeros((16,), jnp.float32)

        for j in range(zc):
            pltpu.sync_copy(
                rows_v.at[0], acc_agg.at[pl.ds(row0 + j * CHUNK, CHUNK)])
        pltpu.sync_copy(z_v, acc_s.at[pl.ds(row0, per_node)])
        cpa.wait()
        cpb.wait()
        plsc.subcore_barrier()

        pltpu.async_copy(hs_hbm.at[src_v.at[0]], rows_v.at[0], sem_g.at[0])
        for j in range(4):
            pltpu.async_copy(dinv_hbm.at[dst_v.at[j]], dval_v.at[j],
                             sem_dg.at[j])

        @pl.loop(0, ngr)
        def _(g):
            for b in range(GB):
                ck = g * GB + b
                br = b % RB
                b1 = (b + 1) % RB
                b4 = (b + 4) % GB

                def _rows_ahead():
                    pltpu.make_async_copy(
                        rows_v.at[b1], acc_agg.at[dst_v.at[0]],
                        sem_s.at[b1]).wait()
                    pltpu.async_copy(hs_hbm.at[src_v.at[ck + 1]],
                                     rows_v.at[b1], sem_g.at[b1])

                if b == 0:
                    @pl.when(g >= 1)
                    def _():
                        pltpu.make_async_copy(
                            rows_v.at[b1], acc_agg.at[dst_v.at[0]],
                            sem_s.at[b1]).wait()
                    pltpu.async_copy(hs_hbm.at[src_v.at[ck + 1]],
                                     rows_v.at[b1], sem_g.at[b1])
                elif b < GB - 1:
                    _rows_ahead()
                else:
                    @pl.when(g < ngr - 1)
                    def _():
                        _rows_ahead()

                pltpu.make_async_copy(
                    hs_hbm.at[src_v.at[ck]], rows_v.at[br],
                    sem_g.at[br]).wait()
                pltpu.async_copy(rows_v.at[br], acc_agg.at[dst_v.at[ck]],
                                 sem_s.at[br], add=True)

                def _scal_ahead():
                    pltpu.make_async_copy(
                        dval_v.at[b4], acc_s.at[src_v.at[0]],
                        sem_ss.at[b4]).wait()
                    pltpu.async_copy(dinv_hbm.at[dst_v.at[ck + 4]],
                                     dval_v.at[b4], sem_dg.at[b4])

                if b < 4:
                    @pl.when(g >= 1)
                    def _():
                        pltpu.make_async_copy(
                            dval_v.at[b4], acc_s.at[src_v.at[0]],
                            sem_ss.at[b4]).wait()
                    pltpu.async_copy(dinv_hbm.at[dst_v.at[ck + 4]],
                                     dval_v.at[b4], sem_dg.at[b4])
                else:
                    @pl.when(g < ngr - 1)
                    def _():
                        _scal_ahead()

                pltpu.make_async_copy(
                    dinv_hbm.at[dst_v.at[ck]], dval_v.at[b],
                    sem_dg.at[b]).wait()
                pltpu.async_copy(dval_v.at[b], acc_s.at[src_v.at[ck]],
                                 sem_ss.at[b], add=True)

        for b in range(RB):
            pltpu.make_async_copy(
                rows_v.at[b], acc_agg.at[dst_v.at[0]], sem_s.at[b]).wait()
        for b in range(GB):
            pltpu.make_async_copy(
                dval_v.at[b], acc_s.at[src_v.at[0]], sem_ss.at[b]).wait()

        plsc.subcore_barrier()
        sl = pl.ds(row0, per_node)
        pltpu.sync_copy(acc_agg.at[sl], agg_hbm.at[cid, sl])
        pltpu.sync_copy(acc_s.at[sl], s_hbm.at[cid, sl])

    return k(hs, dinv_flat, src2d, dst2d)


def _tc_matmul(x, W1, n_pad, blk):
    nb = n_pad // blk
    in_dim = x.shape[1]
    hid = W1.shape[1]

    def body(x_ref, w1_ref, h_ref):
        h_ref[...] = jnp.dot(x_ref[...], w1_ref[...],
                             preferred_element_type=jnp.float32)

    return pl.pallas_call(
        body,
        grid=(nb,),
        in_specs=[
            pl.BlockSpec((blk, in_dim), lambda i: (i, 0)),
            pl.BlockSpec((in_dim, hid), lambda i: (0, 0)),
        ],
        out_specs=pl.BlockSpec((blk, hid), lambda i: (i, 0)),
        out_shape=jax.ShapeDtypeStruct((n_pad, hid), jnp.float32),
    )(x, W1)


def _tc_scale(deg2, h, n, n_pad, blk):
    nb = n_pad // blk
    rr = blk // 128
    hid = h.shape[1]

    def body(deg_ref, h_ref, dlane_ref, dcol_ref, hs_ref):
        i = pl.program_id(0)
        d = deg_ref[0] + deg_ref[1] + 1.0
        node = (lax.broadcasted_iota(jnp.int32, (rr, 128), 0) * 128
                + lax.broadcasted_iota(jnp.int32, (rr, 128), 1) + i * blk)
        valid = node < n
        dlane = jnp.where(valid, lax.rsqrt(d), 0.0)
        m = jnp.where(valid, 1.0, 0.0)
        dlane_ref[...] = dlane
        for r in range(rr):
            dcol = dlane[r:r + 1, :].T
            mcol = m[r:r + 1, :].T
            sl = pl.ds(r * 128, 128)
            dcol_ref[sl, :] = dcol
            hs_ref[sl, :] = jnp.where(mcol > 0.0, h_ref[sl, :] * dcol, 0.0)

    return pl.pallas_call(
        body,
        grid=(nb,),
        in_specs=[
            pl.BlockSpec((NC, rr, 128), lambda i: (0, i, 0)),
            pl.BlockSpec((blk, hid), lambda i: (i, 0)),
        ],
        out_specs=[
            pl.BlockSpec((rr, 128), lambda i: (i, 0)),
            pl.BlockSpec((blk, 1), lambda i: (i, 0)),
            pl.BlockSpec((blk, hid), lambda i: (i, 0)),
        ],
        out_shape=[
            jax.ShapeDtypeStruct((n_pad // 128, 128), jnp.float32),
            jax.ShapeDtypeStruct((n_pad, 1), jnp.float32),
            jax.ShapeDtypeStruct((n_pad, hid), jnp.float32),
        ],
    )(deg2, h)


def _tc_final(aggp, hs, dcol, sp2, b1, W2, b2, n, n_pad, blk):
    nb = n_pad // blk
    rr = blk // 128
    hid = hs.shape[1]
    out_dim = W2.shape[1]

    def body(aggp_ref, hs_ref, dcol_ref, sp_ref, b1_ref, w2_ref, b2_ref,
             out_ref, acc_ref):
        i = pl.program_id(0)
        agg = aggp_ref[0] + aggp_ref[1]
        slane = sp_ref[0] + sp_ref[1]
        part = jnp.zeros((1, hid), jnp.float32)
        for r in range(rr):
            sl = pl.ds(r * 128, 128)
            dinv = dcol_ref[sl, :]
            scol = slane[r:r + 1, :].T
            h1 = jnp.maximum(
                dinv * (agg[r * 128:(r + 1) * 128, :] + hs_ref[sl, :])
                + b1_ref[...], 0.0)
            c = dinv * (scol + dinv)
            part = part + jnp.sum(c * h1, axis=0, keepdims=True)

        @pl.when(i == 0)
        def _():
            acc_ref[...] = jnp.zeros_like(acc_ref)

        acc_ref[...] += part

        @pl.when(i == nb - 1)
        def _():
            v = acc_ref[...] * (1.0 / n)
            out_ref[...] = (
                jnp.dot(v, w2_ref[...], preferred_element_type=jnp.float32)
                + b2_ref[...]
            )

    return pl.pallas_call(
        body,
        grid=(nb,),
        in_specs=[
            pl.BlockSpec((NC, blk, hid), lambda i: (0, i, 0)),
            pl.BlockSpec((blk, hid), lambda i: (i, 0)),
            pl.BlockSpec((blk, 1), lambda i: (i, 0)),
            pl.BlockSpec((NC, rr, 128), lambda i: (0, i, 0)),
            pl.BlockSpec((1, hid), lambda i: (0, 0)),
            pl.BlockSpec((hid, out_dim), lambda i: (0, 0)),
            pl.BlockSpec((1, out_dim), lambda i: (0, 0)),
        ],
        out_specs=pl.BlockSpec((1, out_dim), lambda i: (0, 0)),
        out_shape=jax.ShapeDtypeStruct((1, out_dim), jnp.float32),
        scratch_shapes=[pltpu.VMEM((1, hid), jnp.float32)],
    )(aggp, hs, dcol, sp2, b1, W2, b2)


def kernel(x, edge_index, W1, b1, W2, b2):
    n, in_dim = x.shape
    hid = W1.shape[1]
    e = edge_index.shape[1]

    n_pad = -(-n // 2048) * 2048
    egrain = NW * CHUNK * GB
    e_pad = -(-e // egrain) * egrain
    blk = 1024

    src = edge_index[0].astype(jnp.int32)
    dst = edge_index[1].astype(jnp.int32)
    pad_e = e_pad - e
    pad_idx = jnp.asarray(
        n + (np.arange(pad_e, dtype=np.int32) % (n_pad - n)), jnp.int32)
    src_p = jnp.concatenate([src, pad_idx])
    dst_p = jnp.concatenate([dst, pad_idx])
    src2d = src_p.reshape(e_pad // CHUNK, CHUNK)
    dst2d = dst_p.reshape(e_pad // CHUNK, CHUNK)

    degp = _sc_degree(dst2d, n_pad, e_pad)
    h = _tc_matmul(x, W1, n_pad, 2048)
    dlane, dcol, hs = _tc_scale(
        degp.reshape(NC, n_pad // 128, 128), h, n, n_pad, blk)
    aggp, sp = _sc_edge_aggregate(
        hs, dlane.reshape(n_pad), src2d, dst2d, n_pad, e_pad, hid)
    out = _tc_final(
        aggp, hs, dcol, sp.reshape(NC, n_pad // 128, 128),
        b1.reshape(1, hid), W2, b2.reshape(1, -1), n, n_pad, blk)
    return out


# speedup vs baseline: 40.9115x; 1.0491x over previous
import functools

import jax
import jax.numpy as jnp
import numpy as np
from jax import lax
from jax.experimental import pallas as pl
from jax.experimental.pallas import tpu as pltpu
from jax.experimental.pallas import tpu_sc as plsc

NC = 2
NS = 16
NW = NC * NS
CHUNK = 128
GB = 8
RB = 2

_MESH = plsc.VectorSubcoreMesh(core_axis_name="c", subcore_axis_name="s")


def _sc_degree(dst2d, n_pad, e_pad):
    nd = e_pad // (NW * CHUNK)
    per_node = n_pad // NS

    @functools.partial(
        pl.kernel,
        mesh=_MESH,
        out_type=jax.ShapeDtypeStruct((NC, n_pad), jnp.float32),
        scratch_types=[
            pltpu.VMEM((nd, CHUNK), jnp.int32),
            pltpu.VMEM((CHUNK,), jnp.float32),
            pltpu.VMEM((per_node,), jnp.float32),
            pltpu.VMEM_SHARED((n_pad,), jnp.float32),
            pltpu.SemaphoreType.DMA,
            pltpu.SemaphoreType.DMA,
        ],
    )
    def k(dst_hbm, out_hbm, idx_v, ones_v, z_v, acc_sh, sem_i, sem_w):
        cid = lax.axis_index("c")
        sid = lax.axis_index("s")
        wid = sid * NC + cid
        cp = pltpu.async_copy(dst_hbm.at[pl.ds(wid * nd, nd)], idx_v, sem_i)

        @pl.loop(0, CHUNK, step=16)
        def _(i):
            ones_v[pl.ds(i, 16)] = jnp.ones((16,), jnp.float32)

        @pl.loop(0, per_node, step=16)
        def _(i):
            z_v[pl.ds(i, 16)] = jnp.zeros((16,), jnp.float32)

        pltpu.sync_copy(z_v, acc_sh.at[pl.ds(sid * per_node, per_node)])
        cp.wait()
        plsc.subcore_barrier()

        @pl.loop(0, nd // GB)
        def _(g):
            for b in range(GB):
                pltpu.async_copy(
                    ones_v, acc_sh.at[idx_v.at[g * GB + b]], sem_w, add=True)
            for b in range(GB):
                pltpu.make_async_copy(
                    ones_v, acc_sh.at[idx_v.at[0]], sem_w).wait()

        plsc.subcore_barrier()
        sl = pl.ds(sid * per_node, per_node)
        pltpu.sync_copy(acc_sh.at[sl], out_hbm.at[cid, sl])

    return k(dst2d)


def _sc_edge_aggregate(hs, dinv_flat, src2d, dst2d, n_pad, e_pad, hid):
    nd = e_pad // (NW * CHUNK)
    ngr = nd // GB
    per_node = n_pad // NS
    zc = per_node // CHUNK

    @functools.partial(
        pl.kernel,
        mesh=_MESH,
        out_type=(
            jax.ShapeDtypeStruct((NC, n_pad, hid), jnp.float32),
            jax.ShapeDtypeStruct((NC, n_pad), jnp.float32),
        ),
        scratch_types=[
            pltpu.VMEM((nd, CHUNK), jnp.int32),
            pltpu.VMEM((nd, CHUNK), jnp.int32),
            pltpu.VMEM((RB, CHUNK, hid), jnp.float32),
            pltpu.VMEM((GB, CHUNK), jnp.float32),
            pltpu.VMEM((per_node,), jnp.float32),
            pltpu.VMEM_SHARED((n_pad, hid), jnp.float32),
            pltpu.VMEM_SHARED((n_pad,), jnp.float32),
            pltpu.SemaphoreType.DMA((2,)),
            pltpu.SemaphoreType.DMA((RB,)),
            pltpu.SemaphoreType.DMA((RB,)),
            pltpu.SemaphoreType.DMA((GB,)),
            pltpu.SemaphoreType.DMA((GB,)),
        ],
    )
    def k(hs_hbm, dinv_hbm, src_hbm, dst_hbm, agg_hbm, s_hbm,
          src_v, dst_v, rows_v, dval_v, z_v, acc_agg, acc_s,
          sem_i, sem_g, sem_s, sem_dg, sem_ss):
        cid = lax.axis_index("c")
        sid = lax.axis_index("s")
        wid = sid * NC + cid
        row0 = sid * per_node
        c0 = wid * nd

        cpa = pltpu.async_copy(src_hbm.at[pl.ds(c0, nd)], src_v, sem_i.at[0])
        cpb = pltpu.async_copy(dst_hbm.at[pl.ds(c0, nd)], dst_v, sem_i.at[1])

        @pl.loop(0, CHUNK)
        def _(r):
            @pl.loop(0, hid, step=16)
            def _(cc):
                rows_v[0, r, pl.ds(cc, 16)] = jnp.zeros((16,), jnp.float32)

        @pl.loop(0, per_node, step=16)
        def _(i):
            z_v[pl.ds(i, 16)] = jnp.zeros((16,), jnp.float32)

        for j in range(zc):
            pltpu.sync_copy(
                rows_v.at[0], acc_agg.at[pl.ds(row0 + j * CHUNK, CHUNK)])
        pltpu.sync_copy(z_v, acc_s.at[pl.ds(row0, per_node)])
        cpa.wait()
        cpb.wait()
        plsc.subcore_barrier()

        pltpu.async_copy(hs_hbm.at[src_v.at[0]], rows_v.at[0], sem_g.at[0])
        for j in range(4):
            pltpu.async_copy(dinv_hbm.at[dst_v.at[j]], dval_v.at[j],
                             sem_dg.at[j])

        @pl.loop(0, ngr)
        def _(g):
            for b in range(GB):
                ck = g * GB + b
                br = b % RB
                b1 = (b + 1) % RB
                b4 = (b + 4) % GB

                def _rows_ahead():
                    pltpu.make_async_copy(
                        rows_v.at[b1], acc_agg.at[dst_v.at[0]],
                        sem_s.at[b1]).wait()
                    pltpu.async_copy(hs_hbm.at[src_v.at[ck + 1]],
                                     rows_v.at[b1], sem_g.at[b1])

                if b == 0:
                    @pl.when(g >= 1)
                    def _():
                        pltpu.make_async_copy(
                            rows_v.at[b1], acc_agg.at[dst_v.at[0]],
                            sem_s.at[b1]).wait()
                    pltpu.async_copy(hs_hbm.at[src_v.at[ck + 1]],
                                     rows_v.at[b1], sem_g.at[b1])
                elif b < GB - 1:
                    _rows_ahead()
                else:
                    @pl.when(g < ngr - 1)
                    def _():
                        _rows_ahead()

                pltpu.make_async_copy(
                    hs_hbm.at[src_v.at[ck]], rows_v.at[br],
                    sem_g.at[br]).wait()
                pltpu.async_copy(rows_v.at[br], acc_agg.at[dst_v.at[ck]],
                                 sem_s.at[br], add=True)

                def _scal_ahead():
                    pltpu.make_async_copy(
                        dval_v.at[b4], acc_s.at[src_v.at[0]],
                        sem_ss.at[b4]).wait()
                    pltpu.async_copy(dinv_hbm.at[dst_v.at[ck + 4]],
                                     dval_v.at[b4], sem_dg.at[b4])

                if b < 4:
                    @pl.when(g >= 1)
                    def _():
                        pltpu.make_async_copy(
                            dval_v.at[b4], acc_s.at[src_v.at[0]],
                            sem_ss.at[b4]).wait()
                    pltpu.async_copy(dinv_hbm.at[dst_v.at[ck + 4]],
                                     dval_v.at[b4], sem_dg.at[b4])
                else:
                    @pl.when(g < ngr - 1)
                    def _():
                        _scal_ahead()

                pltpu.make_async_copy(
                    dinv_hbm.at[dst_v.at[ck]], dval_v.at[b],
                    sem_dg.at[b]).wait()
                pltpu.async_copy(dval_v.at[b], acc_s.at[src_v.at[ck]],
                                 sem_ss.at[b], add=True)

        for b in range(RB):
            pltpu.make_async_copy(
                rows_v.at[b], acc_agg.at[dst_v.at[0]], sem_s.at[b]).wait()
        for b in range(GB):
            pltpu.make_async_copy(
                dval_v.at[b], acc_s.at[src_v.at[0]], sem_ss.at[b]).wait()

        plsc.subcore_barrier()
        sl = pl.ds(row0, per_node)
        pltpu.sync_copy(acc_agg.at[sl], agg_hbm.at[cid, sl])
        pltpu.sync_copy(acc_s.at[sl], s_hbm.at[cid, sl])

    return k(hs, dinv_flat, src2d, dst2d)


def _tc_pad_edges(edge_index, n, n_pad, e, e_pad):
    eb = 40960
    nb = e_pad // eb

    def body(ei_ref, out_ref):
        i = pl.program_id(0)
        col = (lax.broadcasted_iota(jnp.int32, (2, eb), 1) + i * eb)
        pad = n + lax.rem(col - e, n_pad - n)
        out_ref[...] = jnp.where(col < e, ei_ref[...], pad)

    return pl.pallas_call(
        body,
        grid=(nb,),
        in_specs=[pl.BlockSpec((2, eb), lambda i: (0, i))],
        out_specs=pl.BlockSpec((2, eb), lambda i: (0, i)),
        out_shape=jax.ShapeDtypeStruct((2, e_pad), jnp.int32),
    )(edge_index)


def _tc_matmul(x, W1, n_pad, blk):
    nb = n_pad // blk
    in_dim = x.shape[1]
    hid = W1.shape[1]

    def body(x_ref, w1_ref, h_ref):
        h_ref[...] = jnp.dot(x_ref[...], w1_ref[...],
                             preferred_element_type=jnp.float32)

    return pl.pallas_call(
        body,
        grid=(nb,),
        in_specs=[
            pl.BlockSpec((blk, in_dim), lambda i: (i, 0)),
            pl.BlockSpec((in_dim, hid), lambda i: (0, 0)),
        ],
        out_specs=pl.BlockSpec((blk, hid), lambda i: (i, 0)),
        out_shape=jax.ShapeDtypeStruct((n_pad, hid), jnp.float32),
    )(x, W1)


def _tc_scale(deg2, h, n, n_pad, blk):
    nb = n_pad // blk
    rr = blk // 128
    hid = h.shape[1]

    def body(deg_ref, h_ref, dlane_ref, hs_ref):
        i = pl.program_id(0)
        d = deg_ref[0] + deg_ref[1] + 1.0
        node = (lax.broadcasted_iota(jnp.int32, (rr, 128), 0) * 128
                + lax.broadcasted_iota(jnp.int32, (rr, 128), 1) + i * blk)
        valid = node < n
        dlane = jnp.where(valid, lax.rsqrt(d), 0.0)
        m = jnp.where(valid, 1.0, 0.0)
        dlane_ref[...] = dlane
        for r in range(rr):
            dcol = dlane[r:r + 1, :].T
            mcol = m[r:r + 1, :].T
            sl = pl.ds(r * 128, 128)
            hs_ref[sl, :] = jnp.where(mcol > 0.0, h_ref[sl, :] * dcol, 0.0)

    return pl.pallas_call(
        body,
        grid=(nb,),
        in_specs=[
            pl.BlockSpec((NC, rr, 128), lambda i: (0, i, 0)),
            pl.BlockSpec((blk, hid), lambda i: (i, 0)),
        ],
        out_specs=[
            pl.BlockSpec((rr, 128), lambda i: (i, 0)),
            pl.BlockSpec((blk, hid), lambda i: (i, 0)),
        ],
        out_shape=[
            jax.ShapeDtypeStruct((n_pad // 128, 128), jnp.float32),
            jax.ShapeDtypeStruct((n_pad, hid), jnp.float32),
        ],
    )(deg2, h)


def _tc_final(aggp, hs, dlane, sp2, b1, W2, b2, n, n_pad, blk):
    nb = n_pad // blk
    rr = blk // 128
    hid = hs.shape[1]
    out_dim = W2.shape[1]

    def body(aggp_ref, hs_ref, dlane_ref, sp_ref, b1_ref, w2_ref, b2_ref,
             out_ref, acc_ref):
        i = pl.program_id(0)
        agg = aggp_ref[0] + aggp_ref[1]
        slane = sp_ref[0] + sp_ref[1]
        dl = dlane_ref[...]
        part = jnp.zeros((1, hid), jnp.float32)
        for r in range(rr):
            sl = pl.ds(r * 128, 128)
            dinv = dl[r:r + 1, :].T
            scol = slane[r:r + 1, :].T
            h1 = jnp.maximum(
                dinv * (agg[r * 128:(r + 1) * 128, :] + hs_ref[sl, :])
                + b1_ref[...], 0.0)
            c = dinv * (scol + dinv)
            part = part + jnp.sum(c * h1, axis=0, keepdims=True)

        @pl.when(i == 0)
        def _():
            acc_ref[...] = jnp.zeros_like(acc_ref)

        acc_ref[...] += part

        @pl.when(i == nb - 1)
        def _():
            v = acc_ref[...] * (1.0 / n)
            out_ref[...] = (
                jnp.dot(v, w2_ref[...], preferred_element_type=jnp.float32)
                + b2_ref[...]
            )

    return pl.pallas_call(
        body,
        grid=(nb,),
        in_specs=[
            pl.BlockSpec((NC, blk, hid), lambda i: (0, i, 0)),
            pl.BlockSpec((blk, hid), lambda i: (i, 0)),
            pl.BlockSpec((rr, 128), lambda i: (i, 0)),
            pl.BlockSpec((NC, rr, 128), lambda i: (0, i, 0)),
            pl.BlockSpec((1, hid), lambda i: (0, 0)),
            pl.BlockSpec((hid, out_dim), lambda i: (0, 0)),
            pl.BlockSpec((1, out_dim), lambda i: (0, 0)),
        ],
        out_specs=pl.BlockSpec((1, out_dim), lambda i: (0, 0)),
        out_shape=jax.ShapeDtypeStruct((1, out_dim), jnp.float32),
        scratch_shapes=[pltpu.VMEM((1, hid), jnp.float32)],
    )(aggp, hs, dlane, sp2, b1, W2, b2)


def kernel(x, edge_index, W1, b1, W2, b2):
    n, in_dim = x.shape
    hid = W1.shape[1]
    e = edge_index.shape[1]

    n_pad = -(-n // 2048) * 2048
    egrain = NW * CHUNK * GB
    e_pad = -(-e // egrain) * egrain
    blk = 1024

    ei_p = _tc_pad_edges(edge_index.astype(jnp.int32), n, n_pad, e, e_pad)
    src2d = ei_p[0].reshape(e_pad // CHUNK, CHUNK)
    dst2d = ei_p[1].reshape(e_pad // CHUNK, CHUNK)

    degp = _sc_degree(dst2d, n_pad, e_pad)
    h = _tc_matmul(x, W1, n_pad, 2048)
    dlane, hs = _tc_scale(
        degp.reshape(NC, n_pad // 128, 128), h, n, n_pad, blk)
    aggp, sp = _sc_edge_aggregate(
        hs, dlane.reshape(n_pad), src2d, dst2d, n_pad, e_pad, hid)
    out = _tc_final(
        aggp, hs, dlane, sp.reshape(NC, n_pad // 128, 128),
        b1.reshape(1, hid), W2, b2.reshape(1, -1), n, n_pad, blk)
    return out


# speedup vs baseline: 42.3691x; 1.0356x over previous
import functools

import jax
import jax.numpy as jnp
import numpy as np
from jax import lax
from jax.experimental import pallas as pl
from jax.experimental.pallas import tpu as pltpu
from jax.experimental.pallas import tpu_sc as plsc

NC = 2
NS = 16
NW = NC * NS
CHUNK = 128
GB = 8
RB = 2

_MESH = plsc.VectorSubcoreMesh(core_axis_name="c", subcore_axis_name="s")


def _sc_degree(ei2d, n_pad, e_pad):
    nd = e_pad // (NW * CHUNK)
    ec = e_pad // CHUNK
    per_node = n_pad // NS

    @functools.partial(
        pl.kernel,
        mesh=_MESH,
        out_type=jax.ShapeDtypeStruct((NC, n_pad), jnp.float32),
        scratch_types=[
            pltpu.VMEM((nd, CHUNK), jnp.int32),
            pltpu.VMEM((CHUNK,), jnp.float32),
            pltpu.VMEM((per_node,), jnp.float32),
            pltpu.VMEM_SHARED((n_pad,), jnp.float32),
            pltpu.SemaphoreType.DMA,
            pltpu.SemaphoreType.DMA,
        ],
    )
    def k(ei_hbm, out_hbm, idx_v, ones_v, z_v, acc_sh, sem_i, sem_w):
        cid = lax.axis_index("c")
        sid = lax.axis_index("s")
        wid = sid * NC + cid
        cp = pltpu.async_copy(
            ei_hbm.at[pl.ds(ec + wid * nd, nd)], idx_v, sem_i)

        @pl.loop(0, CHUNK, step=16)
        def _(i):
            ones_v[pl.ds(i, 16)] = jnp.ones((16,), jnp.float32)

        @pl.loop(0, per_node, step=16)
        def _(i):
            z_v[pl.ds(i, 16)] = jnp.zeros((16,), jnp.float32)

        pltpu.sync_copy(z_v, acc_sh.at[pl.ds(sid * per_node, per_node)])
        cp.wait()
        plsc.subcore_barrier()

        @pl.loop(0, nd // GB)
        def _(g):
            for b in range(GB):
                pltpu.async_copy(
                    ones_v, acc_sh.at[idx_v.at[g * GB + b]], sem_w, add=True)
            for b in range(GB):
                pltpu.make_async_copy(
                    ones_v, acc_sh.at[idx_v.at[0]], sem_w).wait()

        plsc.subcore_barrier()
        sl = pl.ds(sid * per_node, per_node)
        pltpu.sync_copy(acc_sh.at[sl], out_hbm.at[cid, sl])

    return k(ei2d)


def _sc_edge_aggregate(hs, dinv_flat, ei2d, n_pad, e_pad, hid):
    nd = e_pad // (NW * CHUNK)
    ec = e_pad // CHUNK
    ngr = nd // GB
    per_node = n_pad // NS
    zc = per_node // CHUNK

    @functools.partial(
        pl.kernel,
        mesh=_MESH,
        out_type=(
            jax.ShapeDtypeStruct((NC, n_pad, hid), jnp.float32),
            jax.ShapeDtypeStruct((NC, n_pad), jnp.float32),
        ),
        scratch_types=[
            pltpu.VMEM((nd, CHUNK), jnp.int32),
            pltpu.VMEM((nd, CHUNK), jnp.int32),
            pltpu.VMEM((RB, CHUNK, hid), jnp.float32),
            pltpu.VMEM((GB, CHUNK), jnp.float32),
            pltpu.VMEM((per_node,), jnp.float32),
            pltpu.VMEM_SHARED((n_pad, hid), jnp.float32),
            pltpu.VMEM_SHARED((n_pad,), jnp.float32),
            pltpu.SemaphoreType.DMA((2,)),
            pltpu.SemaphoreType.DMA((RB,)),
            pltpu.SemaphoreType.DMA((RB,)),
            pltpu.SemaphoreType.DMA((GB,)),
            pltpu.SemaphoreType.DMA((GB,)),
        ],
    )
    def k(hs_hbm, dinv_hbm, ei_hbm, agg_hbm, s_hbm,
          src_v, dst_v, rows_v, dval_v, z_v, acc_agg, acc_s,
          sem_i, sem_g, sem_s, sem_dg, sem_ss):
        cid = lax.axis_index("c")
        sid = lax.axis_index("s")
        wid = sid * NC + cid
        row0 = sid * per_node
        c0 = wid * nd

        cpa = pltpu.async_copy(ei_hbm.at[pl.ds(c0, nd)], src_v, sem_i.at[0])
        cpb = pltpu.async_copy(
            ei_hbm.at[pl.ds(ec + c0, nd)], dst_v, sem_i.at[1])

        @pl.loop(0, CHUNK)
        def _(r):
            @pl.loop(0, hid, step=16)
            def _(cc):
                rows_v[0, r, pl.ds(cc, 16)] = jnp.zeros((16,), jnp.float32)

        @pl.loop(0, per_node, step=16)
        def _(i):
            z_v[pl.ds(i, 16)] = jnp.zeros((16,), jnp.float32)

        for j in range(zc):
            pltpu.sync_copy(
                rows_v.at[0], acc_agg.at[pl.ds(row0 + j * CHUNK, CHUNK)])
        pltpu.sync_copy(z_v, acc_s.at[pl.ds(row0, per_node)])
        cpa.wait()
        cpb.wait()
        plsc.subcore_barrier()

        pltpu.async_copy(hs_hbm.at[src_v.at[0]], rows_v.at[0], sem_g.at[0])
        for j in range(4):
            pltpu.async_copy(dinv_hbm.at[dst_v.at[j]], dval_v.at[j],
                             sem_dg.at[j])

        @pl.loop(0, ngr)
        def _(g):
            for b in range(GB):
                ck = g * GB + b
                br = b % RB
                b1 = (b + 1) % RB
                b4 = (b + 4) % GB

                def _rows_ahead():
                    pltpu.make_async_copy(
                        rows_v.at[b1], acc_agg.at[dst_v.at[0]],
                        sem_s.at[b1]).wait()
                    pltpu.async_copy(hs_hbm.at[src_v.at[ck + 1]],
                                     rows_v.at[b1], sem_g.at[b1])

                if b == 0:
                    @pl.when(g >= 1)
                    def _():
                        pltpu.make_async_copy(
                            rows_v.at[b1], acc_agg.at[dst_v.at[0]],
                            sem_s.at[b1]).wait()
                    pltpu.async_copy(hs_hbm.at[src_v.at[ck + 1]],
                                     rows_v.at[b1], sem_g.at[b1])
                elif b < GB - 1:
                    _rows_ahead()
                else:
                    @pl.when(g < ngr - 1)
                    def _():
                        _rows_ahead()

                pltpu.make_async_copy(
                    hs_hbm.at[src_v.at[ck]], rows_v.at[br],
                    sem_g.at[br]).wait()
                pltpu.async_copy(rows_v.at[br], acc_agg.at[dst_v.at[ck]],
                                 sem_s.at[br], add=True)

                def _scal_ahead():
                    pltpu.make_async_copy(
                        dval_v.at[b4], acc_s.at[src_v.at[0]],
                        sem_ss.at[b4]).wait()
                    pltpu.async_copy(dinv_hbm.at[dst_v.at[ck + 4]],
                                     dval_v.at[b4], sem_dg.at[b4])

                if b < 4:
                    @pl.when(g >= 1)
                    def _():
                        pltpu.make_async_copy(
                            dval_v.at[b4], acc_s.at[src_v.at[0]],
                            sem_ss.at[b4]).wait()
                    pltpu.async_copy(dinv_hbm.at[dst_v.at[ck + 4]],
                                     dval_v.at[b4], sem_dg.at[b4])
                else:
                    @pl.when(g < ngr - 1)
                    def _():
                        _scal_ahead()

                pltpu.make_async_copy(
                    dinv_hbm.at[dst_v.at[ck]], dval_v.at[b],
                    sem_dg.at[b]).wait()
                pltpu.async_copy(dval_v.at[b], acc_s.at[src_v.at[ck]],
                                 sem_ss.at[b], add=True)

        for b in range(RB):
            pltpu.make_async_copy(
                rows_v.at[b], acc_agg.at[dst_v.at[0]], sem_s.at[b]).wait()
        for b in range(GB):
            pltpu.make_async_copy(
                dval_v.at[b], acc_s.at[src_v.at[0]], sem_ss.at[b]).wait()

        plsc.subcore_barrier()
        sl = pl.ds(row0, per_node)
        pltpu.sync_copy(acc_agg.at[sl], agg_hbm.at[cid, sl])
        pltpu.sync_copy(acc_s.at[sl], s_hbm.at[cid, sl])

    return k(hs, dinv_flat, ei2d)


def _tc_pad_edges(edge_index, n, n_pad, e, e_pad):
    eb = 40960
    nb = e_pad // eb

    def body(ei_ref, out_ref):
        i = pl.program_id(0)
        col = (lax.broadcasted_iota(jnp.int32, (2, eb), 1) + i * eb)
        pad = n + lax.rem(col - e, n_pad - n)
        out_ref[...] = jnp.where(col < e, ei_ref[...], pad)

    return pl.pallas_call(
        body,
        grid=(nb,),
        in_specs=[pl.BlockSpec((2, eb), lambda i: (0, i))],
        out_specs=pl.BlockSpec((2, eb), lambda i: (0, i)),
        out_shape=jax.ShapeDtypeStruct((2, e_pad), jnp.int32),
    )(edge_index)


def _tc_matmul(x, W1, n_pad, blk):
    nb = n_pad // blk
    in_dim = x.shape[1]
    hid = W1.shape[1]

    def body(x_ref, w1_ref, h_ref):
        h_ref[...] = jnp.dot(x_ref[...], w1_ref[...],
                             preferred_element_type=jnp.float32)

    return pl.pallas_call(
        body,
        grid=(nb,),
        in_specs=[
            pl.BlockSpec((blk, in_dim), lambda i: (i, 0)),
            pl.BlockSpec((in_dim, hid), lambda i: (0, 0)),
        ],
        out_specs=pl.BlockSpec((blk, hid), lambda i: (i, 0)),
        out_shape=jax.ShapeDtypeStruct((n_pad, hid), jnp.float32),
    )(x, W1)


def _tc_scale(deg2, h, n, n_pad, blk):
    nb = n_pad // blk
    rr = blk // 128
    hid = h.shape[1]

    def body(deg_ref, h_ref, dlane_ref, hs_ref):
        i = pl.program_id(0)
        d = deg_ref[0] + deg_ref[1] + 1.0
        node = (lax.broadcasted_iota(jnp.int32, (rr, 128), 0) * 128
                + lax.broadcasted_iota(jnp.int32, (rr, 128), 1) + i * blk)
        valid = node < n
        dlane = jnp.where(valid, lax.rsqrt(d), 0.0)
        m = jnp.where(valid, 1.0, 0.0)
        dlane_ref[...] = dlane
        for r in range(rr):
            dcol = dlane[r:r + 1, :].T
            mcol = m[r:r + 1, :].T
            sl = pl.ds(r * 128, 128)
            hs_ref[sl, :] = jnp.where(mcol > 0.0, h_ref[sl, :] * dcol, 0.0)

    return pl.pallas_call(
        body,
        grid=(nb,),
        in_specs=[
            pl.BlockSpec((NC, rr, 128), lambda i: (0, i, 0)),
            pl.BlockSpec((blk, hid), lambda i: (i, 0)),
        ],
        out_specs=[
            pl.BlockSpec((rr, 128), lambda i: (i, 0)),
            pl.BlockSpec((blk, hid), lambda i: (i, 0)),
        ],
        out_shape=[
            jax.ShapeDtypeStruct((n_pad // 128, 128), jnp.float32),
            jax.ShapeDtypeStruct((n_pad, hid), jnp.float32),
        ],
    )(deg2, h)


def _tc_final(aggp, hs, dlane, sp2, b1, W2, b2, n, n_pad, blk):
    nb = n_pad // blk
    rr = blk // 128
    hid = hs.shape[1]
    out_dim = W2.shape[1]

    def body(aggp_ref, hs_ref, dlane_ref, sp_ref, b1_ref, w2_ref, b2_ref,
             out_ref, acc_ref):
        i = pl.program_id(0)
        agg = aggp_ref[0] + aggp_ref[1]
        slane = sp_ref[0] + sp_ref[1]
        dl = dlane_ref[...]
        part = jnp.zeros((1, hid), jnp.float32)
        for r in range(rr):
            sl = pl.ds(r * 128, 128)
            dinv = dl[r:r + 1, :].T
            scol = slane[r:r + 1, :].T
            h1 = jnp.maximum(
                dinv * (agg[r * 128:(r + 1) * 128, :] + hs_ref[sl, :])
                + b1_ref[...], 0.0)
            c = dinv * (scol + dinv)
            part = part + jnp.sum(c * h1, axis=0, keepdims=True)

        @pl.when(i == 0)
        def _():
            acc_ref[...] = jnp.zeros_like(acc_ref)

        acc_ref[...] += part

        @pl.when(i == nb - 1)
        def _():
            v = acc_ref[...] * (1.0 / n)
            out_ref[...] = (
                jnp.dot(v, w2_ref[...], preferred_element_type=jnp.float32)
                + b2_ref[...]
            )

    return pl.pallas_call(
        body,
        grid=(nb,),
        in_specs=[
            pl.BlockSpec((NC, blk, hid), lambda i: (0, i, 0)),
            pl.BlockSpec((blk, hid), lambda i: (i, 0)),
            pl.BlockSpec((rr, 128), lambda i: (i, 0)),
            pl.BlockSpec((NC, rr, 128), lambda i: (0, i, 0)),
            pl.BlockSpec((1, hid), lambda i: (0, 0)),
            pl.BlockSpec((hid, out_dim), lambda i: (0, 0)),
            pl.BlockSpec((1, out_dim), lambda i: (0, 0)),
        ],
        out_specs=pl.BlockSpec((1, out_dim), lambda i: (0, 0)),
        out_shape=jax.ShapeDtypeStruct((1, out_dim), jnp.float32),
        scratch_shapes=[pltpu.VMEM((1, hid), jnp.float32)],
    )(aggp, hs, dlane, sp2, b1, W2, b2)


def kernel(x, edge_index, W1, b1, W2, b2):
    n, in_dim = x.shape
    hid = W1.shape[1]
    e = edge_index.shape[1]

    n_pad = -(-n // 2048) * 2048
    egrain = NW * CHUNK * GB
    e_pad = -(-e // egrain) * egrain
    blk = 2048

    ei_p = _tc_pad_edges(edge_index.astype(jnp.int32), n, n_pad, e, e_pad)
    ei2d = ei_p.reshape(2 * (e_pad // CHUNK), CHUNK)

    degp = _sc_degree(ei2d, n_pad, e_pad)
    h = _tc_matmul(x, W1, n_pad, 2048)
    dlane, hs = _tc_scale(
        degp.reshape(NC, n_pad // 128, 128), h, n, n_pad, blk)
    aggp, sp = _sc_edge_aggregate(
        hs, dlane.reshape(n_pad), ei2d, n_pad, e_pad, hid)
    out = _tc_final(
        aggp, hs, dlane, sp.reshape(NC, n_pad // 128, 128),
        b1.reshape(1, hid), W2, b2.reshape(1, -1), n, n_pad, blk)
    return out


# speedup vs baseline: 43.0681x; 1.0165x over previous
import functools

import jax
import jax.numpy as jnp
import numpy as np
from jax import lax
from jax.experimental import pallas as pl
from jax.experimental.pallas import tpu as pltpu
from jax.experimental.pallas import tpu_sc as plsc

NC = 2
NS = 16
NW = NC * NS
CHUNK = 128
GB = 8
RB = 2

_MESH = plsc.VectorSubcoreMesh(core_axis_name="c", subcore_axis_name="s")


def _load_idx(ei_hbm, tail_hbm, row, idx_v, wid, nd, ec_real):
    bw = ec_real // nd

    @pl.when(wid != bw)
    def _():
        pltpu.sync_copy(ei_hbm.at[row, pl.ds(wid * nd, nd)], idx_v)

    @pl.when(wid == bw)
    def _():
        pltpu.sync_copy(tail_hbm.at[row], idx_v)


def _sc_degree(ei3, tail3, n, n_pad, e, e_pad):
    nd = e_pad // (NW * CHUNK)
    ec_real = e // CHUNK
    per_node = n_pad // NS

    @functools.partial(
        pl.kernel,
        mesh=_MESH,
        out_type=jax.ShapeDtypeStruct((NC, n_pad), jnp.float32),
        scratch_types=[
            pltpu.VMEM((nd, CHUNK), jnp.int32),
            pltpu.VMEM((CHUNK,), jnp.float32),
            pltpu.VMEM((per_node,), jnp.float32),
            pltpu.VMEM_SHARED((n_pad,), jnp.float32),
            pltpu.SemaphoreType.DMA,
        ],
    )
    def k(ei_hbm, tail_hbm, out_hbm, idx_v, ones_v, z_v, acc_sh, sem_w):
        cid = lax.axis_index("c")
        sid = lax.axis_index("s")
        wid = sid * NC + cid
        _load_idx(ei_hbm, tail_hbm, 1, idx_v, wid, nd, ec_real)

        @pl.loop(0, CHUNK, step=16)
        def _(i):
            ones_v[pl.ds(i, 16)] = jnp.ones((16,), jnp.float32)

        @pl.loop(0, per_node, step=16)
        def _(i):
            z_v[pl.ds(i, 16)] = jnp.zeros((16,), jnp.float32)

        pltpu.sync_copy(z_v, acc_sh.at[pl.ds(sid * per_node, per_node)])
        plsc.subcore_barrier()

        @pl.loop(0, nd // GB)
        def _(g):
            for b in range(GB):
                pltpu.async_copy(
                    ones_v, acc_sh.at[idx_v.at[g * GB + b]], sem_w, add=True)
            for b in range(GB):
                pltpu.make_async_copy(
                    ones_v, acc_sh.at[idx_v.at[0]], sem_w).wait()

        plsc.subcore_barrier()
        sl = pl.ds(sid * per_node, per_node)
        pltpu.sync_copy(acc_sh.at[sl], out_hbm.at[cid, sl])

    return k(ei3, tail3)


def _sc_edge_aggregate(hs, dinv_flat, ei3, tail3, n, n_pad, e, e_pad, hid):
    nd = e_pad // (NW * CHUNK)
    ec_real = e // CHUNK
    ngr = nd // GB
    per_node = n_pad // NS
    zc = per_node // CHUNK

    @functools.partial(
        pl.kernel,
        mesh=_MESH,
        out_type=(
            jax.ShapeDtypeStruct((NC, n_pad, hid), jnp.float32),
            jax.ShapeDtypeStruct((NC, n_pad), jnp.float32),
        ),
        scratch_types=[
            pltpu.VMEM((nd, CHUNK), jnp.int32),
            pltpu.VMEM((nd, CHUNK), jnp.int32),
            pltpu.VMEM((RB, CHUNK, hid), jnp.float32),
            pltpu.VMEM((GB, CHUNK), jnp.float32),
            pltpu.VMEM((per_node,), jnp.float32),
            pltpu.VMEM_SHARED((n_pad, hid), jnp.float32),
            pltpu.VMEM_SHARED((n_pad,), jnp.float32),
            pltpu.SemaphoreType.DMA((RB,)),
            pltpu.SemaphoreType.DMA((RB,)),
            pltpu.SemaphoreType.DMA((GB,)),
            pltpu.SemaphoreType.DMA((GB,)),
        ],
    )
    def k(hs_hbm, dinv_hbm, ei_hbm, tail_hbm, agg_hbm, s_hbm,
          src_v, dst_v, rows_v, dval_v, z_v, acc_agg, acc_s,
          sem_g, sem_s, sem_dg, sem_ss):
        cid = lax.axis_index("c")
        sid = lax.axis_index("s")
        wid = sid * NC + cid
        row0 = sid * per_node

        _load_idx(ei_hbm, tail_hbm, 0, src_v, wid, nd, ec_real)
        _load_idx(ei_hbm, tail_hbm, 1, dst_v, wid, nd, ec_real)

        @pl.loop(0, CHUNK)
        def _(r):
            @pl.loop(0, hid, step=16)
            def _(cc):
                rows_v[0, r, pl.ds(cc, 16)] = jnp.zeros((16,), jnp.float32)

        @pl.loop(0, per_node, step=16)
        def _(i):
            z_v[pl.ds(i, 16)] = jnp.zeros((16,), jnp.float32)

        for j in range(zc):
            pltpu.sync_copy(
                rows_v.at[0], acc_agg.at[pl.ds(row0 + j * CHUNK, CHUNK)])
        pltpu.sync_copy(z_v, acc_s.at[pl.ds(row0, per_node)])
        plsc.subcore_barrier()

        pltpu.async_copy(hs_hbm.at[src_v.at[0]], rows_v.at[0], sem_g.at[0])
        for j in range(4):
            pltpu.async_copy(dinv_hbm.at[dst_v.at[j]], dval_v.at[j],
                             sem_dg.at[j])

        @pl.loop(0, ngr)
        def _(g):
            for b in range(GB):
                ck = g * GB + b
                br = b % RB
                b1 = (b + 1) % RB
                b4 = (b + 4) % GB

                def _rows_ahead():
                    pltpu.make_async_copy(
                        rows_v.at[b1], acc_agg.at[dst_v.at[0]],
                        sem_s.at[b1]).wait()
                    pltpu.async_copy(hs_hbm.at[src_v.at[ck + 1]],
                                     rows_v.at[b1], sem_g.at[b1])

                if b == 0:
                    @pl.when(g >= 1)
                    def _():
                        pltpu.make_async_copy(
                            rows_v.at[b1], acc_agg.at[dst_v.at[0]],
                            sem_s.at[b1]).wait()
                    pltpu.async_copy(hs_hbm.at[src_v.at[ck + 1]],
                                     rows_v.at[b1], sem_g.at[b1])
                elif b < GB - 1:
                    _rows_ahead()
                else:
                    @pl.when(g < ngr - 1)
                    def _():
                        _rows_ahead()

                pltpu.make_async_copy(
                    hs_hbm.at[src_v.at[ck]], rows_v.at[br],
                    sem_g.at[br]).wait()
                pltpu.async_copy(rows_v.at[br], acc_agg.at[dst_v.at[ck]],
                                 sem_s.at[br], add=True)

                def _scal_ahead():
                    pltpu.make_async_copy(
                        dval_v.at[b4], acc_s.at[src_v.at[0]],
                        sem_ss.at[b4]).wait()
                    pltpu.async_copy(dinv_hbm.at[dst_v.at[ck + 4]],
                                     dval_v.at[b4], sem_dg.at[b4])

                if b < 4:
                    @pl.when(g >= 1)
                    def _():
                        pltpu.make_async_copy(
                            dval_v.at[b4], acc_s.at[src_v.at[0]],
                            sem_ss.at[b4]).wait()
                    pltpu.async_copy(dinv_hbm.at[dst_v.at[ck + 4]],
                                     dval_v.at[b4], sem_dg.at[b4])
                else:
                    @pl.when(g < ngr - 1)
                    def _():
                        _scal_ahead()

                pltpu.make_async_copy(
                    dinv_hbm.at[dst_v.at[ck]], dval_v.at[b],
                    sem_dg.at[b]).wait()
                pltpu.async_copy(dval_v.at[b], acc_s.at[src_v.at[ck]],
                                 sem_ss.at[b], add=True)

        for b in range(RB):
            pltpu.make_async_copy(
                rows_v.at[b], acc_agg.at[dst_v.at[0]], sem_s.at[b]).wait()
        for b in range(GB):
            pltpu.make_async_copy(
                dval_v.at[b], acc_s.at[src_v.at[0]], sem_ss.at[b]).wait()

        plsc.subcore_barrier()
        sl = pl.ds(row0, per_node)
        pltpu.sync_copy(acc_agg.at[sl], agg_hbm.at[cid, sl])
        pltpu.sync_copy(acc_s.at[sl], s_hbm.at[cid, sl])

    return k(hs, dinv_flat, ei3, tail3)


def _tc_matmul(x, W1, n_pad, blk):
    nb = n_pad // blk
    in_dim = x.shape[1]
    hid = W1.shape[1]

    def body(x_ref, w1_ref, h_ref):
        h_ref[...] = jnp.dot(x_ref[...], w1_ref[...],
                             preferred_element_type=jnp.float32)

    return pl.pallas_call(
        body,
        grid=(nb,),
        in_specs=[
            pl.BlockSpec((blk, in_dim), lambda i: (i, 0)),
            pl.BlockSpec((in_dim, hid), lambda i: (0, 0)),
        ],
        out_specs=pl.BlockSpec((blk, hid), lambda i: (i, 0)),
        out_shape=jax.ShapeDtypeStruct((n_pad, hid), jnp.float32),
    )(x, W1)


def _tc_scale(deg2, h, n, n_pad, blk):
    nb = n_pad // blk
    rr = blk // 128
    hid = h.shape[1]

    def body(deg_ref, h_ref, dlane_ref, hs_ref):
        i = pl.program_id(0)
        d = deg_ref[0] + deg_ref[1] + 1.0
        node = (lax.broadcasted_iota(jnp.int32, (rr, 128), 0) * 128
                + lax.broadcasted_iota(jnp.int32, (rr, 128), 1) + i * blk)
        valid = node < n
        dlane = jnp.where(valid, lax.rsqrt(d), 0.0)
        m = jnp.where(valid, 1.0, 0.0)
        dlane_ref[...] = dlane
        for r in range(rr):
            dcol = dlane[r:r + 1, :].T
            mcol = m[r:r + 1, :].T
            sl = pl.ds(r * 128, 128)
            hs_ref[sl, :] = jnp.where(mcol > 0.0, h_ref[sl, :] * dcol, 0.0)

    return pl.pallas_call(
        body,
        grid=(nb,),
        in_specs=[
            pl.BlockSpec((NC, rr, 128), lambda i: (0, i, 0)),
            pl.BlockSpec((blk, hid), lambda i: (i, 0)),
        ],
        out_specs=[
            pl.BlockSpec((rr, 128), lambda i: (i, 0)),
            pl.BlockSpec((blk, hid), lambda i: (i, 0)),
        ],
        out_shape=[
            jax.ShapeDtypeStruct((n_pad // 128, 128), jnp.float32),
            jax.ShapeDtypeStruct((n_pad, hid), jnp.float32),
        ],
    )(deg2, h)


def _tc_final(aggp, hs, dlane, sp2, b1, W2, b2, n, n_pad, blk):
    nb = n_pad // blk
    rr = blk // 128
    hid = hs.shape[1]
    out_dim = W2.shape[1]

    def body(aggp_ref, hs_ref, dlane_ref, sp_ref, b1_ref, w2_ref, b2_ref,
             out_ref, acc_ref):
        i = pl.program_id(0)
        agg = aggp_ref[0] + aggp_ref[1]
        slane = sp_ref[0] + sp_ref[1]
        dl = dlane_ref[...]
        part = jnp.zeros((1, hid), jnp.float32)
        for r in range(rr):
            sl = pl.ds(r * 128, 128)
            dinv = dl[r:r + 1, :].T
            scol = slane[r:r + 1, :].T
            h1 = jnp.maximum(
                dinv * (agg[r * 128:(r + 1) * 128, :] + hs_ref[sl, :])
                + b1_ref[...], 0.0)
            c = dinv * (scol + dinv)
            part = part + jnp.sum(c * h1, axis=0, keepdims=True)

        @pl.when(i == 0)
        def _():
            acc_ref[...] = jnp.zeros_like(acc_ref)

        acc_ref[...] += part

        @pl.when(i == nb - 1)
        def _():
            v = acc_ref[...] * (1.0 / n)
            out_ref[...] = (
                jnp.dot(v, w2_ref[...], preferred_element_type=jnp.float32)
                + b2_ref[...]
            )

    return pl.pallas_call(
        body,
        grid=(nb,),
        in_specs=[
            pl.BlockSpec((NC, blk, hid), lambda i: (0, i, 0)),
            pl.BlockSpec((blk, hid), lambda i: (i, 0)),
            pl.BlockSpec((rr, 128), lambda i: (i, 0)),
            pl.BlockSpec((NC, rr, 128), lambda i: (0, i, 0)),
            pl.BlockSpec((1, hid), lambda i: (0, 0)),
            pl.BlockSpec((hid, out_dim), lambda i: (0, 0)),
            pl.BlockSpec((1, out_dim), lambda i: (0, 0)),
        ],
        out_specs=pl.BlockSpec((1, out_dim), lambda i: (0, 0)),
        out_shape=jax.ShapeDtypeStruct((1, out_dim), jnp.float32),
        scratch_shapes=[pltpu.VMEM((1, hid), jnp.float32)],
    )(aggp, hs, dlane, sp2, b1, W2, b2)


def kernel(x, edge_index, W1, b1, W2, b2):
    n, in_dim = x.shape
    hid = W1.shape[1]
    e = edge_index.shape[1]

    n_pad = -(-n // 2048) * 2048
    egrain = NW * CHUNK * GB
    e_pad = -(-e // egrain) * egrain
    blk = 2048

    ei = edge_index.astype(jnp.int32)
    ei3 = ei.reshape(2, e // CHUNK, CHUNK)
    nd = e_pad // (NW * CHUNK)
    bw = (e // CHUNK) // nd
    pad_idx = jnp.asarray(
        n + (np.arange(e_pad - e, dtype=np.int32) % (n_pad - n)), jnp.int32)
    tail3 = jnp.concatenate(
        [ei[:, bw * nd * CHUNK:],
         jnp.broadcast_to(pad_idx[None, :], (2, e_pad - e))],
        axis=1).reshape(2, nd, CHUNK)

    degp = _sc_degree(ei3, tail3, n, n_pad, e, e_pad)
    h = _tc_matmul(x, W1, n_pad, 2048)
    dlane, hs = _tc_scale(
        degp.reshape(NC, n_pad // 128, 128), h, n, n_pad, blk)
    aggp, sp = _sc_edge_aggregate(
        hs, dlane.reshape(n_pad), ei3, tail3, n, n_pad, e, e_pad, hid)
    out = _tc_final(
        aggp, hs, dlane, sp.reshape(NC, n_pad // 128, 128),
        b1.reshape(1, hid), W2, b2.reshape(1, -1), n, n_pad, blk)
    return out
